# baseline (speedup 1.0000x reference)
import functools

import jax
import jax.numpy as jnp
from jax import lax
from jax.experimental import pallas as pl
from jax.experimental.pallas import tpu as pltpu
from jax.experimental.pallas import tpu_sc as plsc

B = 4
S = 2048
D = 1024
DFF = 4096
E = 8
CAP = 320

NTOK = B * S
SLOTS_PER_E = B * CAP
NSLOT = E * SLOTS_PER_E
NSLOT_PAD = 9 * SLOTS_PER_E
FBLK = 1024

NW = 32
TPW = NTOK // NW
CH = 128
DI = D // 2


def _router_body(x_ref, wr_ref, logits_ref, maxp_ref, scat_ref, gath_ref,
                 routed_ref, eidx_ref, h16_ref):
    b = pl.program_id(0)
    x = x_ref[0]
    wr = wr_ref[...]
    logits = lax.dot_general(x, wr, (((1,), (1,)), ((), ())),
                             preferred_element_type=jnp.float32)
    m = jnp.max(logits, axis=-1, keepdims=True)
    eexp = jnp.exp(logits - m)
    probs = eexp / jnp.sum(eexp, axis=-1, keepdims=True)
    maxp = jnp.max(probs, axis=-1, keepdims=True)
    iota_e = lax.broadcasted_iota(jnp.int32, (S, E), 1)
    ei = jnp.min(jnp.where(probs >= maxp, iota_e, E), axis=-1, keepdims=True)
    onehot = (iota_e == ei).astype(jnp.int32)
    prio = onehot
    sh = 1
    while sh < S:
        prio = prio + jnp.concatenate(
            [jnp.zeros((sh, E), jnp.int32), prio[:S - sh]], axis=0)
        sh *= 2
    pchosen = jnp.sum(prio * onehot, axis=-1, keepdims=True)
    routed = pchosen <= CAP
    slot = (ei * B + b) * CAP + (pchosen - 1)
    logits_ref[0] = logits
    maxp_ref[0] = maxp
    scat_ref[0] = jnp.where(routed, slot, NSLOT)
    gath_ref[0] = jnp.where(routed, slot, 0)
    routed_ref[0] = routed.astype(jnp.float32)
    eidx_ref[0] = jnp.where(routed, ei, 0)
    h16_ref[0] = x.astype(jnp.bfloat16)


def _router(hidden_states, Wr):
    return pl.pallas_call(
        _router_body,
        grid=(B,),
        in_specs=[
            pl.BlockSpec((1, S, D), lambda b: (b, 0, 0)),
            pl.BlockSpec((E, D), lambda b: (0, 0)),
        ],
        out_specs=[
            pl.BlockSpec((1, S, E), lambda b: (b, 0, 0)),
            pl.BlockSpec((1, S, 1), lambda b: (b, 0, 0)),
            pl.BlockSpec((1, S, 1), lambda b: (b, 0, 0)),
            pl.BlockSpec((1, S, 1), lambda b: (b, 0, 0)),
            pl.BlockSpec((1, S, 1), lambda b: (b, 0, 0)),
            pl.BlockSpec((1, S, 1), lambda b: (b, 0, 0)),
            pl.BlockSpec((1, S, D), lambda b: (b, 0, 0)),
        ],
        out_shape=[
            jax.ShapeDtypeStruct((B, S, E), jnp.float32),
            jax.ShapeDtypeStruct((B, S, 1), jnp.float32),
            jax.ShapeDtypeStruct((B, S, 1), jnp.int32),
            jax.ShapeDtypeStruct((B, S, 1), jnp.int32),
            jax.ShapeDtypeStruct((B, S, 1), jnp.float32),
            jax.ShapeDtypeStruct((B, S, 1), jnp.int32),
            jax.ShapeDtypeStruct((B, S, D), jnp.bfloat16),
        ],
    )(hidden_states, Wr)


@functools.cache
def _sc_kernels():
    mesh = plsc.VectorSubcoreMesh(core_axis_name="c", subcore_axis_name="s")

    @functools.partial(
        pl.kernel,
        mesh=mesh,
        out_type=jax.ShapeDtypeStruct((NSLOT_PAD, DI), jnp.int32),
        scratch_types=[
            pltpu.VMEM((CH,), jnp.int32),
            pltpu.VMEM((CH, DI), jnp.int32),
            pltpu.SemaphoreType.DMA,
        ],
    )
    def _dispatch(h_hbm, scat_hbm, xs_hbm, idx_v, rows_v, sem):
        wid = lax.axis_index("s") * 2 + lax.axis_index("c")
        for k in range(TPW // CH):
            base = wid * TPW + k * CH
            pltpu.sync_copy(scat_hbm.at[pl.ds(base, CH)], idx_v)
            pltpu.sync_copy(h_hbm.at[pl.ds(base, CH)], rows_v)
            pltpu.async_copy(rows_v, xs_hbm.at[idx_v], sem).wait()

    @functools.partial(
        pl.kernel,
        mesh=mesh,
        out_type=jax.ShapeDtypeStruct((NTOK, DI), jnp.int32),
        scratch_types=[
            pltpu.VMEM((CH,), jnp.int32),
            pltpu.VMEM((CH, DI), jnp.int32),
            pltpu.SemaphoreType.DMA,
        ],
    )
    def _gather(ys_hbm, gath_hbm, yt_hbm, idx_v, rows_v, sem):
        wid = lax.axis_index("s") * 2 + lax.axis_index("c")
        for k in range(TPW // CH):
            base = wid * TPW + k * CH
            pltpu.sync_copy(gath_hbm.at[pl.ds(base, CH)], idx_v)
            pltpu.async_copy(ys_hbm.at[idx_v], rows_v, sem).wait()
            pltpu.sync_copy(rows_v, yt_hbm.at[pl.ds(base, CH)])

    return _dispatch, _gather


def _ffn_body(x_ref, wi_ref, wo_ref, o_ref, acc_ref):
    f = pl.program_id(1)
    x = x_ref[0]
    wi = wi_ref[0].astype(jnp.bfloat16)
    wo = wo_ref[0].astype(jnp.bfloat16)
    h = lax.dot_general(x, wi, (((1,), (1,)), ((), ())),
                        preferred_element_type=jnp.float32)
    hb = jnp.maximum(h, 0.0).astype(jnp.bfloat16)
    y = lax.dot_general(hb, wo, (((1,), (1,)), ((), ())),
                        preferred_element_type=jnp.float32)

    @pl.when(f == 0)
    def _():
        acc_ref[...] = y

    @pl.when(f > 0)
    def _():
        acc_ref[...] += y

    @pl.when(f == DFF // FBLK - 1)
    def _():
        o_ref[0] = acc_ref[...].astype(jnp.bfloat16)


def _ffn(xs, Wi, Wo):
    return pl.pallas_call(
        _ffn_body,
        grid=(E, DFF // FBLK),
        in_specs=[
            pl.BlockSpec((1, SLOTS_PER_E, D), lambda e, f: (e, 0, 0)),
            pl.BlockSpec((1, FBLK, D), lambda e, f: (e, f, 0)),
            pl.BlockSpec((1, D, FBLK), lambda e, f: (e, 0, f)),
        ],
        out_specs=pl.BlockSpec((1, SLOTS_PER_E, D), lambda e, f: (e, 0, 0)),
        out_shape=jax.ShapeDtypeStruct((E, SLOTS_PER_E, D), jnp.bfloat16),
        scratch_shapes=[pltpu.VMEM((SLOTS_PER_E, D), jnp.float32)],
    )(xs, Wi, Wo)


def _combine_body(h_ref, y_ref, mp_ref, rt_ref, o_ref):
    sel = jnp.where(rt_ref[...] > 0.0, y_ref[...].astype(jnp.float32),
                    h_ref[...])
    o_ref[...] = mp_ref[...] * sel


def _combine(h2, yt, maxp, routed):
    blk = 256
    return pl.pallas_call(
        _combine_body,
        grid=(NTOK // blk,),
        in_specs=[
            pl.BlockSpec((blk, D), lambda i: (i, 0)),
            pl.BlockSpec((blk, D), lambda i: (i, 0)),
            pl.BlockSpec((blk, 1), lambda i: (i, 0)),
            pl.BlockSpec((blk, 1), lambda i: (i, 0)),
        ],
        out_specs=pl.BlockSpec((blk, D), lambda i: (i, 0)),
        out_shape=jax.ShapeDtypeStruct((NTOK, D), jnp.float32),
    )(h2, yt, maxp, routed)


def kernel(hidden_states, Wr, Wi, Wo):
    h2 = hidden_states.reshape(NTOK, D)
    logits, maxp, scat, gath, routed, eidx, h16 = _router(hidden_states, Wr)
    dispatch, gather = _sc_kernels()
    h16i = lax.bitcast_convert_type(
        h16.reshape(NTOK, DI, 2), jnp.int32).reshape(NTOK, DI)
    xs = dispatch(h16i, scat.reshape(NTOK))
    xs16 = lax.bitcast_convert_type(
        xs.reshape(NSLOT_PAD, DI, 1), jnp.bfloat16).reshape(
            9, SLOTS_PER_E, D)
    ys = _ffn(xs16, Wi, Wo)
    ysi = lax.bitcast_convert_type(
        ys.reshape(NSLOT, DI, 2), jnp.int32).reshape(NSLOT, DI)
    yt = gather(ysi, gath.reshape(NTOK))
    yt16 = lax.bitcast_convert_type(
        yt.reshape(NTOK, DI, 1), jnp.bfloat16).reshape(NTOK, D)
    out = _combine(h2, yt16, maxp.reshape(NTOK, 1), routed.reshape(NTOK, 1))
    return (out.reshape(B, S, D), logits, eidx.reshape(B, S))


# speedup vs baseline: 3.2064x; 3.2064x over previous
import functools

import jax
import jax.numpy as jnp
from jax import lax
from jax.experimental import pallas as pl
from jax.experimental.pallas import tpu as pltpu
from jax.experimental.pallas import tpu_sc as plsc

B = 4
S = 2048
D = 1024
DFF = 4096
E = 8
CAP = 320

NTOK = B * S
SLOTS_PER_E = B * CAP
NSLOT = E * SLOTS_PER_E
NSLOT_PAD = 9 * SLOTS_PER_E
FBLK = 1024

NW = 32
TPW = NTOK // NW
CH = 64


def _router_body(x_ref, wr_ref, logits_ref, maxp_ref, scat_ref, gath_ref,
                 routed_ref, eidx_ref):
    b = pl.program_id(0)
    x = x_ref[0]
    wr = wr_ref[...]
    logits = lax.dot_general(x, wr, (((1,), (1,)), ((), ())),
                             preferred_element_type=jnp.float32)
    m = jnp.max(logits, axis=-1, keepdims=True)
    eexp = jnp.exp(logits - m)
    probs = eexp / jnp.sum(eexp, axis=-1, keepdims=True)
    maxp = jnp.max(probs, axis=-1, keepdims=True)
    iota_e = lax.broadcasted_iota(jnp.int32, (S, E), 1)
    ei = jnp.min(jnp.where(probs >= maxp, iota_e, E), axis=-1, keepdims=True)
    onehot = (iota_e == ei).astype(jnp.int32)
    prio = onehot
    sh = 1
    while sh < S:
        prio = prio + jnp.concatenate(
            [jnp.zeros((sh, E), jnp.int32), prio[:S - sh]], axis=0)
        sh *= 2
    pchosen = jnp.sum(prio * onehot, axis=-1, keepdims=True)
    routed = pchosen <= CAP
    slot = (ei * B + b) * CAP + (pchosen - 1)
    logits_ref[0] = logits
    maxp_ref[0] = maxp
    scat_ref[0] = jnp.where(routed, slot, NSLOT)
    gath_ref[0] = jnp.where(routed, slot, 0)
    routed_ref[0] = routed.astype(jnp.float32)
    eidx_ref[0] = jnp.where(routed, ei, 0)


def _router(hidden_states, Wr):
    return pl.pallas_call(
        _router_body,
        grid=(B,),
        in_specs=[
            pl.BlockSpec((1, S, D), lambda b: (b, 0, 0)),
            pl.BlockSpec((E, D), lambda b: (0, 0)),
        ],
        out_specs=[
            pl.BlockSpec((1, S, E), lambda b: (b, 0, 0)),
            pl.BlockSpec((1, S, 1), lambda b: (b, 0, 0)),
            pl.BlockSpec((1, S, 1), lambda b: (b, 0, 0)),
            pl.BlockSpec((1, S, 1), lambda b: (b, 0, 0)),
            pl.BlockSpec((1, S, 1), lambda b: (b, 0, 0)),
            pl.BlockSpec((1, S, 1), lambda b: (b, 0, 0)),
        ],
        out_shape=[
            jax.ShapeDtypeStruct((B, S, E), jnp.float32),
            jax.ShapeDtypeStruct((B, S, 1), jnp.float32),
            jax.ShapeDtypeStruct((B, S, 1), jnp.int32),
            jax.ShapeDtypeStruct((B, S, 1), jnp.int32),
            jax.ShapeDtypeStruct((B, S, 1), jnp.float32),
            jax.ShapeDtypeStruct((B, S, 1), jnp.int32),
        ],
    )(hidden_states, Wr)


@functools.cache
def _sc_kernels():
    mesh = plsc.VectorSubcoreMesh(core_axis_name="c", subcore_axis_name="s")

    @functools.partial(
        pl.kernel,
        mesh=mesh,
        out_type=jax.ShapeDtypeStruct((NSLOT_PAD, D), jnp.float32),
        scratch_types=[
            pltpu.VMEM((CH,), jnp.int32),
            pltpu.VMEM((CH, D), jnp.float32),
            pltpu.SemaphoreType.DMA,
        ],
    )
    def _dispatch(h_hbm, scat_hbm, xs_hbm, idx_v, rows_v, sem):
        wid = lax.axis_index("s") * 2 + lax.axis_index("c")
        for k in range(TPW // CH):
            base = wid * TPW + k * CH
            pltpu.sync_copy(scat_hbm.at[pl.ds(base, CH)], idx_v)
            pltpu.sync_copy(h_hbm.at[pl.ds(base, CH)], rows_v)
            pltpu.async_copy(rows_v, xs_hbm.at[idx_v], sem).wait()

    @functools.partial(
        pl.kernel,
        mesh=mesh,
        out_type=jax.ShapeDtypeStruct((NTOK, D), jnp.float32),
        scratch_types=[
            pltpu.VMEM((CH,), jnp.int32),
            pltpu.VMEM((CH, D), jnp.float32),
            pltpu.SemaphoreType.DMA,
        ],
    )
    def _gather(ys_hbm, gath_hbm, yt_hbm, idx_v, rows_v, sem):
        wid = lax.axis_index("s") * 2 + lax.axis_index("c")
        for k in range(TPW // CH):
            base = wid * TPW + k * CH
            pltpu.sync_copy(gath_hbm.at[pl.ds(base, CH)], idx_v)
            pltpu.async_copy(ys_hbm.at[idx_v], rows_v, sem).wait()
            pltpu.sync_copy(rows_v, yt_hbm.at[pl.ds(base, CH)])

    return _dispatch, _gather


def _ffn_body(x_ref, wi_ref, wo_ref, o_ref):
    f = pl.program_id(1)
    x = x_ref[0]
    wi = wi_ref[0]
    wo = wo_ref[0]
    h = lax.dot_general(x, wi, (((1,), (1,)), ((), ())),
                        preferred_element_type=jnp.float32)
    hb = jnp.maximum(h, 0.0).astype(jnp.bfloat16)
    y = lax.dot_general(hb, wo.astype(jnp.bfloat16), (((1,), (1,)), ((), ())),
                        preferred_element_type=jnp.float32)

    @pl.when(f == 0)
    def _():
        o_ref[0] = y

    @pl.when(f > 0)
    def _():
        o_ref[0] += y


def _ffn(xs, Wi, Wo):
    return pl.pallas_call(
        _ffn_body,
        grid=(E, DFF // FBLK),
        in_specs=[
            pl.BlockSpec((1, SLOTS_PER_E, D), lambda e, f: (e, 0, 0)),
            pl.BlockSpec((1, FBLK, D), lambda e, f: (e, f, 0)),
            pl.BlockSpec((1, D, FBLK), lambda e, f: (e, 0, f)),
        ],
        out_specs=pl.BlockSpec((1, SLOTS_PER_E, D), lambda e, f: (e, 0, 0)),
        out_shape=jax.ShapeDtypeStruct((E, SLOTS_PER_E, D), jnp.float32),
    )(xs, Wi, Wo)


def _combine_body(h_ref, y_ref, mp_ref, rt_ref, o_ref):
    sel = jnp.where(rt_ref[...] > 0.0, y_ref[...], h_ref[...])
    o_ref[...] = mp_ref[...] * sel


def _combine(h2, yt, maxp, routed):
    blk = 1024
    return pl.pallas_call(
        _combine_body,
        grid=(NTOK // blk,),
        in_specs=[
            pl.BlockSpec((blk, D), lambda i: (i, 0)),
            pl.BlockSpec((blk, D), lambda i: (i, 0)),
            pl.BlockSpec((blk, 1), lambda i: (i, 0)),
            pl.BlockSpec((blk, 1), lambda i: (i, 0)),
        ],
        out_specs=pl.BlockSpec((blk, D), lambda i: (i, 0)),
        out_shape=jax.ShapeDtypeStruct((NTOK, D), jnp.float32),
    )(h2, yt, maxp, routed)


def kernel(hidden_states, Wr, Wi, Wo):
    h2 = hidden_states.reshape(NTOK, D)
    logits, maxp, scat, gath, routed, eidx = _router(hidden_states, Wr)
    dispatch, gather = _sc_kernels()
    xs = dispatch(h2, scat.reshape(NTOK))
    ys = _ffn(xs.reshape(9, SLOTS_PER_E, D), Wi, Wo)
    yt = gather(ys.reshape(NSLOT, D), gath.reshape(NTOK))
    out = _combine(h2, yt, maxp.reshape(NTOK, 1), routed.reshape(NTOK, 1))
    return (out.reshape(B, S, D), logits, eidx.reshape(B, S))
